# Initial kernel scaffold; baseline (speedup 1.0000x reference)
import jax
import jax.numpy as jnp
from jax.experimental import pallas as pl


def kernel(feat_i, U_feat, cnt, W_user, b_user, W_key, W_last, W_e, segment_ids, last_nodes):
    raise NotImplementedError("write your pallas kernel here")


# speedup vs baseline: 5.1841x; 5.1841x over previous
import functools

import jax
import jax.numpy as jnp
from jax import lax
from jax.experimental import pallas as pl
from jax.experimental.pallas import tpu as pltpu
from jax.experimental.pallas import tpu_sc as plsc

NC = 2
NS = 16
NW = NC * NS
CHUNK = 128


def _sc_mesh():
    return plsc.VectorSubcoreMesh(
        core_axis_name="c", subcore_axis_name="s", num_cores=NC, num_subcores=NS
    )


def _sc_gather(table, idx):
    M = idx.shape[0]
    T, D = table.shape
    per_w = M // NW
    n_chunks = per_w // CHUNK
    assert per_w % CHUNK == 0 and M % NW == 0

    @functools.partial(
        pl.kernel,
        out_type=jax.ShapeDtypeStruct((M, D), jnp.float32),
        mesh=_sc_mesh(),
        compiler_params=pltpu.CompilerParams(use_tc_tiling_on_sc=False),
        scratch_types=[
            pltpu.VMEM((CHUNK,), jnp.int32),
            pltpu.VMEM((CHUNK, D), jnp.float32),
            pltpu.SemaphoreType.DMA,
        ],
    )
    def k(table_hbm, idx_hbm, out_hbm, idx_v, rows_v, sem):
        wid = lax.axis_index("s") * NC + lax.axis_index("c")

        def body(i, carry):
            base = pl.multiple_of(wid * per_w + i * CHUNK, CHUNK)
            pltpu.sync_copy(idx_hbm.at[pl.ds(base, CHUNK)], idx_v)
            pltpu.async_copy(table_hbm.at[idx_v], rows_v, sem).wait()
            pltpu.sync_copy(rows_v, out_hbm.at[pl.ds(base, CHUNK)])
            return carry

        lax.fori_loop(0, n_chunks, body, 0)

    return k(table, idx)


def _sc_segment_scatter_add(prod, seg):
    n, W = prod.shape
    B = 4096
    per_w = n // NW
    n_chunks = per_w // CHUNK
    assert n % NW == 0 and per_w % CHUNK == 0
    rows_per_tile = B // NS

    @functools.partial(
        pl.kernel,
        out_type=jax.ShapeDtypeStruct((NC, B, W), jnp.float32),
        mesh=_sc_mesh(),
        compiler_params=pltpu.CompilerParams(use_tc_tiling_on_sc=False),
        scratch_types=[
            pltpu.VMEM((CHUNK,), jnp.int32),
            pltpu.VMEM((CHUNK, W), jnp.float32),
            pltpu.VMEM((16, W), jnp.float32),
            pltpu.VMEM_SHARED((B, W), jnp.float32),
            pltpu.SemaphoreType.DMA,
        ],
    )
    def k(prod_hbm, seg_hbm, out_hbm, seg_v, buf_v, zrow_v, acc_sh, sem):
        cid = lax.axis_index("c")
        sid = lax.axis_index("s")
        wid = sid * NC + cid

        for r in range(16):
            for j in range(W // 16):
                zrow_v[r, pl.ds(j * 16, 16)] = jnp.zeros((16,), jnp.float32)

        def zbody(i, carry):
            row0 = sid * rows_per_tile + i * 16
            pltpu.sync_copy(zrow_v, acc_sh.at[pl.ds(row0, 16)])
            return carry

        lax.fori_loop(0, rows_per_tile // 16, zbody, 0)
        plsc.subcore_barrier()

        def body(i, carry):
            base = pl.multiple_of(wid * per_w + i * CHUNK, CHUNK)
            pltpu.sync_copy(seg_hbm.at[pl.ds(base, CHUNK)], seg_v)
            pltpu.sync_copy(prod_hbm.at[pl.ds(base, CHUNK)], buf_v)
            pltpu.sync_copy(buf_v, acc_sh.at[seg_v], add=True)
            return carry

        lax.fori_loop(0, n_chunks, body, 0)
        plsc.subcore_barrier()

        row0 = sid * rows_per_tile
        pltpu.sync_copy(
            acc_sh.at[pl.ds(row0, rows_per_tile)],
            out_hbm.at[cid, pl.ds(row0, rows_per_tile)],
        )

    return k(prod, seg)


def _tc_query(U_feat, last_rows, W_user, b_user, W_last):
    def body(u_ref, lr_ref, wu_ref, bu_ref, wl_ref, out_ref):
        q = jnp.dot(u_ref[...], wu_ref[...].T, preferred_element_type=jnp.float32)
        q += jnp.dot(lr_ref[...], wl_ref[...].T, preferred_element_type=jnp.float32)
        out_ref[...] = q + bu_ref[...]

    B, D = U_feat.shape
    return pl.pallas_call(
        body,
        out_shape=jax.ShapeDtypeStruct((B, D), jnp.float32),
    )(U_feat, last_rows, W_user, b_user.reshape(1, D), W_last)


def _tc_main(feat, Qb, cnt2, W_key, W_e):
    n, D = feat.shape
    CB = 2048
    grid = n // CB

    def body(f_ref, qb_ref, c_ref, wk_ref, we_ref, out_ref):
        f = f_ref[...]
        kv = jnp.dot(f, wk_ref[...].T, preferred_element_type=jnp.float32)
        z = qb_ref[...] + kv
        sg = 1.0 / (1.0 + jnp.exp(-z))
        a = jnp.dot(sg, we_ref[...].T, preferred_element_type=jnp.float32)
        w = c_ref[...] * jnp.exp(a)
        out_ref[...] = jnp.concatenate(
            [w * f, jnp.broadcast_to(w, (CB, 16))], axis=1
        )

    return pl.pallas_call(
        body,
        grid=(grid,),
        in_specs=[
            pl.BlockSpec((CB, D), lambda i: (i, 0)),
            pl.BlockSpec((CB, D), lambda i: (i, 0)),
            pl.BlockSpec((CB, 1), lambda i: (i, 0)),
            pl.BlockSpec((D, D), lambda i: (0, 0)),
            pl.BlockSpec((1, D), lambda i: (0, 0)),
        ],
        out_specs=pl.BlockSpec((CB, D + 16), lambda i: (i, 0)),
        out_shape=jax.ShapeDtypeStruct((n, D + 16), jnp.float32),
    )(feat, Qb, cnt2, W_key, W_e)


def _tc_final(partials):
    NC_, B, W = partials.shape
    D = 64

    def body(p_ref, out_ref):
        num = p_ref[0, :, 0:D] + p_ref[1, :, 0:D]
        den = p_ref[0, :, D : D + 1] + p_ref[1, :, D : D + 1]
        out_ref[...] = jnp.where(den > 0.0, num / den, 0.0)

    return pl.pallas_call(
        body,
        out_shape=jax.ShapeDtypeStruct((B, D), jnp.float32),
    )(partials)


def kernel(feat_i, U_feat, cnt, W_user, b_user, W_key, W_last, W_e,
           segment_ids, last_nodes):
    n, D = feat_i.shape
    last_rows = _sc_gather(feat_i, last_nodes)
    q = _tc_query(U_feat, last_rows, W_user, b_user, W_last)
    Qb = _sc_gather(q, segment_ids)
    prod = _tc_main(feat_i, Qb, cnt.reshape(n, 1), W_key, W_e)
    partials = _sc_segment_scatter_add(prod, segment_ids)
    return _tc_final(partials)


# speedup vs baseline: 6.5544x; 1.2643x over previous
import functools

import jax
import jax.numpy as jnp
from jax import lax
from jax.experimental import pallas as pl
from jax.experimental.pallas import tpu as pltpu
from jax.experimental.pallas import tpu_sc as plsc

NC = 2
NS = 16
NW = NC * NS
IDXC = 128


def _sc_mesh():
    return plsc.VectorSubcoreMesh(
        core_axis_name="c", subcore_axis_name="s", num_cores=NC, num_subcores=NS
    )


def _sc_params():
    return pltpu.CompilerParams(use_tc_tiling_on_sc=False)


def _sc_gather(table, idx2, M, stage_table):
    T, D = table.shape
    per_w = M // NW
    assert M % NW == 0 and per_w % IDXC == 0
    rows_per_tile_tbl = T // NS

    UR = min(256, per_w)
    units = per_w // UR
    assert per_w % UR == 0 and UR % IDXC == 0
    spu = UR // IDXC
    nb = min(6, units)
    LAG = 2 if nb > 2 else (1 if nb == 2 else 0)
    idx_rows = per_w // IDXC

    scratch = [pltpu.VMEM((idx_rows, IDXC), jnp.int32)]
    scratch += [pltpu.VMEM((UR, D), jnp.float32) for _ in range(nb)]
    scratch += [pltpu.SemaphoreType.DMA for _ in range(2 * nb)]
    if stage_table:
        scratch.append(pltpu.VMEM_SHARED((T, D), jnp.float32))

    @functools.partial(
        pl.kernel,
        out_type=jax.ShapeDtypeStruct((M, D), jnp.float32),
        mesh=_sc_mesh(),
        compiler_params=_sc_params(),
        scratch_types=scratch,
    )
    def k(table_hbm, idx_hbm, out_hbm, *refs):
        idx_v = refs[0]
        bufs = refs[1 : 1 + nb]
        gsem = refs[1 + nb : 1 + 2 * nb]
        wsem = refs[1 + 2 * nb : 1 + 3 * nb]
        cid = lax.axis_index("c")
        sid = lax.axis_index("s")
        wid = sid * NC + cid

        if stage_table:
            tbl = refs[1 + 3 * nb]
            r0 = sid * rows_per_tile_tbl
            pltpu.sync_copy(
                table_hbm.at[pl.ds(r0, rows_per_tile_tbl)],
                tbl.at[pl.ds(r0, rows_per_tile_tbl)],
            )
            plsc.subcore_barrier()
        else:
            tbl = table_hbm

        pltpu.sync_copy(idx_hbm.at[pl.ds(wid * idx_rows, idx_rows)], idx_v)

        gd = {}
        wd = {}
        for t in range(units + LAG):
            b = t % nb
            if t < units:
                if t >= nb:
                    wd.pop(t - nb).wait()
                ds_list = []
                for s in range(spu):
                    d = pltpu.async_copy(
                        tbl.at[idx_v.at[t * spu + s]],
                        bufs[b].at[pl.ds(s * IDXC, IDXC)],
                        gsem[b],
                    )
                    ds_list.append(d)
                gd[t] = ds_list
            u = t - LAG
            if 0 <= u < units:
                bu = u % nb
                for d in gd.pop(u):
                    d.wait()
                base = wid * per_w + u * UR
                wd[u] = pltpu.async_copy(
                    bufs[bu], out_hbm.at[pl.ds(base, UR)], wsem[bu]
                )
        for u in sorted(wd):
            wd[u].wait()

    return k(table, idx2)


def _sc_segment_scatter_add(prod, seg2, B):
    n, W = prod.shape
    per_w = n // NW
    assert n % NW == 0 and per_w % IDXC == 0
    rows_per_tile = B // NS

    UR = min(256, per_w)
    units = per_w // UR
    spu = UR // IDXC
    nb = min(4, units)
    LAG = 1 if nb > 1 else 0
    idx_rows = per_w // IDXC

    scratch = [pltpu.VMEM((idx_rows, IDXC), jnp.int32)]
    scratch += [pltpu.VMEM((UR, W), jnp.float32) for _ in range(nb)]
    scratch.append(pltpu.VMEM((64, W), jnp.float32))
    scratch.append(pltpu.VMEM_SHARED((B, W), jnp.float32))
    scratch += [pltpu.SemaphoreType.DMA for _ in range(2 * nb)]

    @functools.partial(
        pl.kernel,
        out_type=jax.ShapeDtypeStruct((NC, B, W), jnp.float32),
        mesh=_sc_mesh(),
        compiler_params=_sc_params(),
        scratch_types=scratch,
    )
    def k(prod_hbm, seg_hbm, out_hbm, *refs):
        idx_v = refs[0]
        bufs = refs[1 : 1 + nb]
        zrow = refs[1 + nb]
        acc = refs[2 + nb]
        gsem = refs[3 + nb : 3 + 2 * nb]
        ssem = refs[3 + 2 * nb : 3 + 3 * nb]
        cid = lax.axis_index("c")
        sid = lax.axis_index("s")
        wid = sid * NC + cid

        for r in range(64):
            for j in range(W // 16):
                zrow[r, pl.ds(j * 16, 16)] = jnp.zeros((16,), jnp.float32)
        for i in range(rows_per_tile // 64):
            pltpu.sync_copy(zrow, acc.at[pl.ds(sid * rows_per_tile + i * 64, 64)])
        plsc.subcore_barrier()

        pltpu.sync_copy(seg_hbm.at[pl.ds(wid * idx_rows, idx_rows)], idx_v)

        gd = {}
        sd = {}
        for t in range(units + LAG):
            b = t % nb
            if t < units:
                if t >= nb:
                    for d in sd.pop(t - nb):
                        d.wait()
                base = wid * per_w + t * UR
                gd[t] = pltpu.async_copy(
                    prod_hbm.at[pl.ds(base, UR)], bufs[b], gsem[b]
                )
            u = t - LAG
            if 0 <= u < units:
                bu = u % nb
                gd.pop(u).wait()
                ds_list = []
                for s in range(spu):
                    d = pltpu.async_copy(
                        bufs[bu].at[pl.ds(s * IDXC, IDXC)],
                        acc.at[idx_v.at[u * spu + s]],
                        ssem[bu],
                        add=True,
                    )
                    ds_list.append(d)
                sd[u] = ds_list
        for u in sorted(sd):
            for d in sd[u]:
                d.wait()

        plsc.subcore_barrier()
        row0 = sid * rows_per_tile
        pltpu.sync_copy(
            acc.at[pl.ds(row0, rows_per_tile)],
            out_hbm.at[cid, pl.ds(row0, rows_per_tile)],
        )

    return k(prod, seg2)


def _tc_query(U_feat, last_rows, W_user, b_user, W_last):
    def body(u_ref, lr_ref, wu_ref, bu_ref, wl_ref, out_ref):
        q = jnp.dot(u_ref[...], wu_ref[...].T, preferred_element_type=jnp.float32)
        q += jnp.dot(lr_ref[...], wl_ref[...].T, preferred_element_type=jnp.float32)
        out_ref[...] = q + bu_ref[...]

    B, D = U_feat.shape
    return pl.pallas_call(
        body,
        out_shape=jax.ShapeDtypeStruct((B, D), jnp.float32),
    )(U_feat, last_rows, W_user, b_user.reshape(1, D), W_last)


def _tc_main(feat, Qb, cnt2, W_key, W_e):
    n, D = feat.shape
    CB = 2048
    grid = n // CB

    def body(f_ref, qb_ref, c_ref, wk_ref, we_ref, out_ref):
        f = f_ref[...]
        kv = jnp.dot(f, wk_ref[...].T, preferred_element_type=jnp.float32)
        z = qb_ref[...] + kv
        sg = 1.0 / (1.0 + jnp.exp(-z))
        a = jnp.dot(sg, we_ref[...].T, preferred_element_type=jnp.float32)
        w = c_ref[...] * jnp.exp(a)
        out_ref[...] = jnp.concatenate(
            [w * f, jnp.broadcast_to(w, (CB, 16))], axis=1
        )

    return pl.pallas_call(
        body,
        grid=(grid,),
        in_specs=[
            pl.BlockSpec((CB, D), lambda i: (i, 0)),
            pl.BlockSpec((CB, D), lambda i: (i, 0)),
            pl.BlockSpec((CB, 1), lambda i: (i, 0)),
            pl.BlockSpec((D, D), lambda i: (0, 0)),
            pl.BlockSpec((1, D), lambda i: (0, 0)),
        ],
        out_specs=pl.BlockSpec((CB, D + 16), lambda i: (i, 0)),
        out_shape=jax.ShapeDtypeStruct((n, D + 16), jnp.float32),
    )(feat, Qb, cnt2, W_key, W_e)


def _tc_final(partials):
    NC_, B, W = partials.shape
    D = 64

    def body(p_ref, out_ref):
        num = p_ref[0, :, 0:D] + p_ref[1, :, 0:D]
        den = p_ref[0, :, D : D + 1] + p_ref[1, :, D : D + 1]
        out_ref[...] = jnp.where(den > 0.0, num / den, 0.0)

    return pl.pallas_call(
        body,
        out_shape=jax.ShapeDtypeStruct((B, D), jnp.float32),
    )(partials)


def kernel(feat_i, U_feat, cnt, W_user, b_user, W_key, W_last, W_e,
           segment_ids, last_nodes):
    n, D = feat_i.shape
    B = U_feat.shape[0]
    seg2 = segment_ids.reshape(n // IDXC, IDXC)
    last2 = last_nodes.reshape(B // IDXC, IDXC)
    last_rows = _sc_gather(feat_i, last2, B, stage_table=False)
    q = _tc_query(U_feat, last_rows, W_user, b_user, W_last)
    Qb = _sc_gather(q, seg2, n, stage_table=False)
    prod = _tc_main(feat_i, Qb, cnt.reshape(n, 1), W_key, W_e)
    partials = _sc_segment_scatter_add(prod, seg2, B)
    return _tc_final(partials)


# speedup vs baseline: 7.3699x; 1.1244x over previous
import functools

import jax
import jax.numpy as jnp
from jax import lax
from jax.experimental import pallas as pl
from jax.experimental.pallas import tpu as pltpu
from jax.experimental.pallas import tpu_sc as plsc

NC = 2
NS = 16
NW = NC * NS
IDXC = 128


def _sc_mesh():
    return plsc.VectorSubcoreMesh(
        core_axis_name="c", subcore_axis_name="s", num_cores=NC, num_subcores=NS
    )


def _sc_params():
    return pltpu.CompilerParams(use_tc_tiling_on_sc=False)


def _sc_gather(table, idx2, M, stage_table):
    T, D = table.shape
    per_w = M // NW
    assert M % NW == 0 and per_w % IDXC == 0
    rows_per_tile_tbl = T // NS

    UR = min(256, per_w)
    units = per_w // UR
    assert per_w % UR == 0 and UR % IDXC == 0
    spu = UR // IDXC
    nb = min(6, units)
    LAG = 2 if nb > 2 else (1 if nb == 2 else 0)
    idx_rows = per_w // IDXC

    scratch = [pltpu.VMEM((idx_rows, IDXC), jnp.int32)]
    scratch += [pltpu.VMEM((UR, D), jnp.float32) for _ in range(nb)]
    scratch += [pltpu.SemaphoreType.DMA for _ in range(2 * nb)]
    if stage_table:
        scratch.append(pltpu.VMEM_SHARED((T, D), jnp.float32))

    @functools.partial(
        pl.kernel,
        out_type=jax.ShapeDtypeStruct((M, D), jnp.float32),
        mesh=_sc_mesh(),
        compiler_params=_sc_params(),
        scratch_types=scratch,
    )
    def k(table_hbm, idx_hbm, out_hbm, *refs):
        idx_v = refs[0]
        bufs = refs[1 : 1 + nb]
        gsem = refs[1 + nb : 1 + 2 * nb]
        wsem = refs[1 + 2 * nb : 1 + 3 * nb]
        cid = lax.axis_index("c")
        sid = lax.axis_index("s")
        wid = sid * NC + cid

        if stage_table:
            tbl = refs[1 + 3 * nb]
            r0 = sid * rows_per_tile_tbl
            pltpu.sync_copy(
                table_hbm.at[pl.ds(r0, rows_per_tile_tbl)],
                tbl.at[pl.ds(r0, rows_per_tile_tbl)],
            )
            plsc.subcore_barrier()
        else:
            tbl = table_hbm

        pltpu.sync_copy(idx_hbm.at[pl.ds(wid * idx_rows, idx_rows)], idx_v)

        gd = {}
        wd = {}
        for t in range(units + LAG):
            b = t % nb
            if t < units:
                if t >= nb:
                    wd.pop(t - nb).wait()
                ds_list = []
                for s in range(spu):
                    d = pltpu.async_copy(
                        tbl.at[idx_v.at[t * spu + s]],
                        bufs[b].at[pl.ds(s * IDXC, IDXC)],
                        gsem[b],
                    )
                    ds_list.append(d)
                gd[t] = ds_list
            u = t - LAG
            if 0 <= u < units:
                bu = u % nb
                for d in gd.pop(u):
                    d.wait()
                base = wid * per_w + u * UR
                wd[u] = pltpu.async_copy(
                    bufs[bu], out_hbm.at[pl.ds(base, UR)], wsem[bu]
                )
        for u in sorted(wd):
            wd[u].wait()

    return k(table, idx2)


def _sc_segment_scatter_add(prod, seg2, B):
    n, W = prod.shape
    per_w = n // NW
    assert n % NW == 0 and per_w % IDXC == 0
    rows_per_tile = B // NS

    UR = min(256, per_w)
    units = per_w // UR
    spu = UR // IDXC
    nb = min(4, units)
    LAG = 1 if nb > 1 else 0
    idx_rows = per_w // IDXC

    scratch = [pltpu.VMEM((idx_rows, IDXC), jnp.int32)]
    scratch += [pltpu.VMEM((UR, W), jnp.float32) for _ in range(nb)]
    scratch.append(pltpu.VMEM((64, W), jnp.float32))
    scratch.append(pltpu.VMEM_SHARED((B, W), jnp.float32))
    scratch += [pltpu.SemaphoreType.DMA for _ in range(2 * nb)]

    @functools.partial(
        pl.kernel,
        out_type=jax.ShapeDtypeStruct((NC, B, W), jnp.float32),
        mesh=_sc_mesh(),
        compiler_params=_sc_params(),
        scratch_types=scratch,
    )
    def k(prod_hbm, seg_hbm, out_hbm, *refs):
        idx_v = refs[0]
        bufs = refs[1 : 1 + nb]
        zrow = refs[1 + nb]
        acc = refs[2 + nb]
        gsem = refs[3 + nb : 3 + 2 * nb]
        ssem = refs[3 + 2 * nb : 3 + 3 * nb]
        cid = lax.axis_index("c")
        sid = lax.axis_index("s")
        wid = sid * NC + cid

        for r in range(64):
            for j in range(W // 16):
                zrow[r, pl.ds(j * 16, 16)] = jnp.zeros((16,), jnp.float32)
        for i in range(rows_per_tile // 64):
            pltpu.sync_copy(zrow, acc.at[pl.ds(sid * rows_per_tile + i * 64, 64)])
        plsc.subcore_barrier()

        pltpu.sync_copy(seg_hbm.at[pl.ds(wid * idx_rows, idx_rows)], idx_v)

        gd = {}
        sd = {}
        for t in range(units + LAG):
            b = t % nb
            if t < units:
                if t >= nb:
                    for d in sd.pop(t - nb):
                        d.wait()
                base = wid * per_w + t * UR
                gd[t] = pltpu.async_copy(
                    prod_hbm.at[pl.ds(base, UR)], bufs[b], gsem[b]
                )
            u = t - LAG
            if 0 <= u < units:
                bu = u % nb
                gd.pop(u).wait()
                ds_list = []
                for s in range(spu):
                    d = pltpu.async_copy(
                        bufs[bu].at[pl.ds(s * IDXC, IDXC)],
                        acc.at[idx_v.at[u * spu + s]],
                        ssem[bu],
                        add=True,
                    )
                    ds_list.append(d)
                sd[u] = ds_list
        for u in sorted(sd):
            for d in sd[u]:
                d.wait()

        plsc.subcore_barrier()
        row0 = sid * rows_per_tile
        pltpu.sync_copy(
            acc.at[pl.ds(row0, rows_per_tile)],
            out_hbm.at[cid, pl.ds(row0, rows_per_tile)],
        )

    return k(prod, seg2)


def _tc_query(U_feat, last_rows, W_user, b_user, W_last):
    def body(u_ref, lr_ref, wu_ref, bu_ref, wl_ref, out_ref):
        q = jnp.dot(u_ref[...], wu_ref[...].T, preferred_element_type=jnp.float32)
        q += jnp.dot(lr_ref[...], wl_ref[...].T, preferred_element_type=jnp.float32)
        out_ref[...] = q + bu_ref[...]

    B, D = U_feat.shape
    return pl.pallas_call(
        body,
        out_shape=jax.ShapeDtypeStruct((B, D), jnp.float32),
    )(U_feat, last_rows, W_user, b_user.reshape(1, D), W_last)


def _tc_main(feat, Qb, cnt2, W_key, W_e):
    n, D = feat.shape
    CB = 2048
    grid = n // CB

    def body(f_ref, qb_ref, c_ref, wk_ref, we_ref, out_ref):
        f = f_ref[...]
        kv = jnp.dot(f, wk_ref[...].T, preferred_element_type=jnp.float32)
        z = qb_ref[...] + kv
        sg = 1.0 / (1.0 + jnp.exp(-z))
        a = jnp.dot(sg, we_ref[...].T, preferred_element_type=jnp.float32)
        w = c_ref[...] * jnp.exp(a)
        out_ref[...] = jnp.concatenate(
            [w * f, jnp.broadcast_to(w, (CB, 16))], axis=1
        )

    return pl.pallas_call(
        body,
        grid=(grid,),
        in_specs=[
            pl.BlockSpec((CB, D), lambda i: (i, 0)),
            pl.BlockSpec((CB, D), lambda i: (i, 0)),
            pl.BlockSpec((CB, 1), lambda i: (i, 0)),
            pl.BlockSpec((D, D), lambda i: (0, 0)),
            pl.BlockSpec((1, D), lambda i: (0, 0)),
        ],
        out_specs=pl.BlockSpec((CB, D + 16), lambda i: (i, 0)),
        out_shape=jax.ShapeDtypeStruct((n, D + 16), jnp.float32),
    )(feat, Qb, cnt2, W_key, W_e)


def _tc_final(partials):
    NC_, B, W = partials.shape
    D = 64

    def body(p_ref, out_ref):
        num = p_ref[0, :, 0:D] + p_ref[1, :, 0:D]
        den = p_ref[0, :, D : D + 1] + p_ref[1, :, D : D + 1]
        out_ref[...] = jnp.where(den > 0.0, num / den, 0.0)

    return pl.pallas_call(
        body,
        out_shape=jax.ShapeDtypeStruct((B, D), jnp.float32),
    )(partials)


def kernel(feat_i, U_feat, cnt, W_user, b_user, W_key, W_last, W_e,
           segment_ids, last_nodes):
    n, D = feat_i.shape
    B = U_feat.shape[0]
    seg2 = segment_ids.reshape(n // IDXC, IDXC)
    last2 = last_nodes.reshape(B // IDXC, IDXC)
    last_rows = _sc_gather(feat_i, last2, B, stage_table=False)
    q = _tc_query(U_feat, last_rows, W_user, b_user, W_last)
    Qb = _sc_gather(q, seg2, n, stage_table=True)
    prod = _tc_main(feat_i, Qb, cnt.reshape(n, 1), W_key, W_e)
    partials = _sc_segment_scatter_add(prod, seg2, B)
    return _tc_final(partials)


# speedup vs baseline: 9.8861x; 1.3414x over previous
import functools

import jax
import jax.numpy as jnp
from jax import lax
from jax.experimental import pallas as pl
from jax.experimental.pallas import tpu as pltpu
from jax.experimental.pallas import tpu_sc as plsc

NC = 2
NS = 16
NW = NC * NS
IDXC = 128


def _sc_mesh():
    return plsc.VectorSubcoreMesh(
        core_axis_name="c", subcore_axis_name="s", num_cores=NC, num_subcores=NS
    )


def _sc_params():
    return pltpu.CompilerParams(use_tc_tiling_on_sc=False)


def _sc_gather(table, idx2, M, stage_table):
    T, D = table.shape
    per_w = M // NW
    assert M % NW == 0 and per_w % IDXC == 0
    rows_per_tile_tbl = T // NS

    UR = min(256 if D <= 64 else 128, per_w)
    units = per_w // UR
    assert per_w % UR == 0 and UR % IDXC == 0
    spu = UR // IDXC
    nb = min(6 if D <= 64 else 5, units)
    LAG = 2 if nb > 2 else (1 if nb == 2 else 0)
    idx_rows = per_w // IDXC

    scratch = [pltpu.VMEM((idx_rows, IDXC), jnp.int32)]
    scratch += [pltpu.VMEM((UR, D), jnp.float32) for _ in range(nb)]
    scratch += [pltpu.SemaphoreType.DMA for _ in range(2 * nb)]
    if stage_table:
        scratch.append(pltpu.VMEM_SHARED((T, D), jnp.float32))

    @functools.partial(
        pl.kernel,
        out_type=jax.ShapeDtypeStruct((M, D), jnp.float32),
        mesh=_sc_mesh(),
        compiler_params=_sc_params(),
        scratch_types=scratch,
    )
    def k(table_hbm, idx_hbm, out_hbm, *refs):
        idx_v = refs[0]
        bufs = refs[1 : 1 + nb]
        gsem = refs[1 + nb : 1 + 2 * nb]
        wsem = refs[1 + 2 * nb : 1 + 3 * nb]
        cid = lax.axis_index("c")
        sid = lax.axis_index("s")
        wid = sid * NC + cid

        if stage_table:
            tbl = refs[1 + 3 * nb]
            r0 = sid * rows_per_tile_tbl
            pltpu.sync_copy(
                table_hbm.at[pl.ds(r0, rows_per_tile_tbl)],
                tbl.at[pl.ds(r0, rows_per_tile_tbl)],
            )
            plsc.subcore_barrier()
        else:
            tbl = table_hbm

        pltpu.sync_copy(idx_hbm.at[pl.ds(wid * idx_rows, idx_rows)], idx_v)

        gd = {}
        wd = {}
        for t in range(units + LAG):
            b = t % nb
            if t < units:
                if t >= nb:
                    wd.pop(t - nb).wait()
                ds_list = []
                for s in range(spu):
                    d = pltpu.async_copy(
                        tbl.at[idx_v.at[t * spu + s]],
                        bufs[b].at[pl.ds(s * IDXC, IDXC)],
                        gsem[b],
                    )
                    ds_list.append(d)
                gd[t] = ds_list
            u = t - LAG
            if 0 <= u < units:
                bu = u % nb
                for d in gd.pop(u):
                    d.wait()
                base = wid * per_w + u * UR
                wd[u] = pltpu.async_copy(
                    bufs[bu], out_hbm.at[pl.ds(base, UR)], wsem[bu]
                )
        for u in sorted(wd):
            wd[u].wait()

    return k(table, idx2)


def _sc_segment_scatter_add(prod, seg2, B):
    n, W = prod.shape
    per_w = n // NW
    assert n % NW == 0 and per_w % IDXC == 0
    rows_per_tile = B // NS

    UR = min(256 if W <= 64 else 128, per_w)
    units = per_w // UR
    spu = UR // IDXC
    nb = min(4, units)
    LAG = 1 if nb > 1 else 0
    idx_rows = per_w // IDXC

    scratch = [pltpu.VMEM((idx_rows, IDXC), jnp.int32)]
    scratch += [pltpu.VMEM((UR, W), jnp.float32) for _ in range(nb)]
    scratch.append(pltpu.VMEM((64, W), jnp.float32))
    scratch.append(pltpu.VMEM_SHARED((B, W), jnp.float32))
    scratch += [pltpu.SemaphoreType.DMA for _ in range(2 * nb)]

    @functools.partial(
        pl.kernel,
        out_type=jax.ShapeDtypeStruct((NC, B, W), jnp.float32),
        mesh=_sc_mesh(),
        compiler_params=_sc_params(),
        scratch_types=scratch,
    )
    def k(prod_hbm, seg_hbm, out_hbm, *refs):
        idx_v = refs[0]
        bufs = refs[1 : 1 + nb]
        zrow = refs[1 + nb]
        acc = refs[2 + nb]
        gsem = refs[3 + nb : 3 + 2 * nb]
        ssem = refs[3 + 2 * nb : 3 + 3 * nb]
        cid = lax.axis_index("c")
        sid = lax.axis_index("s")
        wid = sid * NC + cid

        for r in range(64):
            for j in range(W // 16):
                zrow[r, pl.ds(j * 16, 16)] = jnp.zeros((16,), jnp.float32)
        for i in range(rows_per_tile // 64):
            pltpu.sync_copy(zrow, acc.at[pl.ds(sid * rows_per_tile + i * 64, 64)])
        plsc.subcore_barrier()

        pltpu.sync_copy(seg_hbm.at[pl.ds(wid * idx_rows, idx_rows)], idx_v)

        gd = {}
        sd = {}
        for t in range(units + LAG):
            b = t % nb
            if t < units:
                if t >= nb:
                    for d in sd.pop(t - nb):
                        d.wait()
                base = wid * per_w + t * UR
                gd[t] = pltpu.async_copy(
                    prod_hbm.at[pl.ds(base, UR)], bufs[b], gsem[b]
                )
            u = t - LAG
            if 0 <= u < units:
                bu = u % nb
                gd.pop(u).wait()
                ds_list = []
                for s in range(spu):
                    d = pltpu.async_copy(
                        bufs[bu].at[pl.ds(s * IDXC, IDXC)],
                        acc.at[idx_v.at[u * spu + s]],
                        ssem[bu],
                        add=True,
                    )
                    ds_list.append(d)
                sd[u] = ds_list
        for u in sorted(sd):
            for d in sd[u]:
                d.wait()

        plsc.subcore_barrier()
        row0 = sid * rows_per_tile
        pltpu.sync_copy(
            acc.at[pl.ds(row0, rows_per_tile)],
            out_hbm.at[cid, pl.ds(row0, rows_per_tile)],
        )

    return k(prod, seg2)


def _tc_query(U_feat, last_rows, W_user, b_user, W_last):
    def body(u_ref, lr_ref, wu_ref, bu_ref, wl_ref, out_ref):
        q = jnp.dot(u_ref[...], wu_ref[...].T, preferred_element_type=jnp.float32)
        q += jnp.dot(lr_ref[...], wl_ref[...].T, preferred_element_type=jnp.float32)
        B, D = u_ref.shape
        out_ref[...] = jnp.concatenate(
            [q + bu_ref[...], jnp.zeros((B, 128 - D), jnp.float32)], axis=1
        )

    B, D = U_feat.shape
    return pl.pallas_call(
        body,
        out_shape=jax.ShapeDtypeStruct((B, 128), jnp.float32),
    )(U_feat, last_rows, W_user, b_user.reshape(1, D), W_last)


def _tc_main(feat, Qb, cnt2, W_key, W_e):
    n, D = feat.shape
    CB = 2048
    grid = n // CB

    def body(f_ref, qb_ref, c_ref, wk_ref, we_ref, out_ref):
        f = f_ref[...]
        kv = jnp.dot(f, wk_ref[...].T, preferred_element_type=jnp.float32)
        z = qb_ref[:, 0:D] + kv
        sg = 1.0 / (1.0 + jnp.exp(-z))
        a = jnp.dot(sg, we_ref[...].T, preferred_element_type=jnp.float32)
        w = c_ref[...] * jnp.exp(a)
        out_ref[...] = jnp.concatenate(
            [w * f, jnp.broadcast_to(w, (CB, 16)),
             jnp.zeros((CB, 48), jnp.float32)], axis=1
        )

    return pl.pallas_call(
        body,
        grid=(grid,),
        in_specs=[
            pl.BlockSpec((CB, D), lambda i: (i, 0)),
            pl.BlockSpec((CB, 128), lambda i: (i, 0)),
            pl.BlockSpec((CB, 1), lambda i: (i, 0)),
            pl.BlockSpec((D, D), lambda i: (0, 0)),
            pl.BlockSpec((1, D), lambda i: (0, 0)),
        ],
        out_specs=pl.BlockSpec((CB, 128), lambda i: (i, 0)),
        out_shape=jax.ShapeDtypeStruct((n, 128), jnp.float32),
    )(feat, Qb, cnt2, W_key, W_e)


def _tc_final(partials):
    NC_, B, W = partials.shape
    D = 64

    def body(p_ref, out_ref):
        num = p_ref[0, :, 0:D] + p_ref[1, :, 0:D]
        den = p_ref[0, :, D : D + 1] + p_ref[1, :, D : D + 1]
        out_ref[...] = jnp.where(den > 0.0, num / den, 0.0)

    return pl.pallas_call(
        body,
        out_shape=jax.ShapeDtypeStruct((B, D), jnp.float32),
    )(partials)


def kernel(feat_i, U_feat, cnt, W_user, b_user, W_key, W_last, W_e,
           segment_ids, last_nodes):
    n, D = feat_i.shape
    B = U_feat.shape[0]
    seg2 = segment_ids.reshape(n // IDXC, IDXC)
    last2 = last_nodes.reshape(B // IDXC, IDXC)
    last_rows = _sc_gather(feat_i, last2, B, stage_table=False)
    q = _tc_query(U_feat, last_rows, W_user, b_user, W_last)
    Qb = _sc_gather(q, seg2, n, stage_table=True)
    prod = _tc_main(feat_i, Qb, cnt.reshape(n, 1), W_key, W_e)
    partials = _sc_segment_scatter_add(prod, seg2, B)
    return _tc_final(partials)
